# baseline (speedup 1.0000x reference)
import functools

import jax
import jax.numpy as jnp
from jax import lax
from jax.experimental import pallas as pl
from jax.experimental.pallas import tpu as pltpu
from jax.experimental.pallas import tpu_sc as plsc

VOCAB = 100000
EMBED = 16
WINDOW = 20
BATCH = 1024
LANES = 128

NUM_CORES = 2
NUM_SUBCORES = 16
NW = NUM_CORES * NUM_SUBCORES
B_PER_W = BATCH // NW
ROWS_PER_W = B_PER_W * WINDOW
GCHUNK = 128
NCHUNK = ROWS_PER_W // GCHUNK

VBLK = 2048
VGRID = (VOCAB + VBLK - 1) // VBLK


def _rsqrt16(s):
    s = jnp.maximum(s, jnp.float32(1e-12))
    i = lax.bitcast_convert_type(s, jnp.int32)
    i = jnp.int32(0x5F3759DF) - (i >> 1)
    y = lax.bitcast_convert_type(i, jnp.float32)
    for _ in range(3):
        y = y * (jnp.float32(1.5) - jnp.float32(0.5) * s * y * y)
    return y


def _sc_pool_body(idx_hbm, table_hbm, out_hbm, idx_v, rows_v, pool_v, sem):
    wid = lax.axis_index("s") * NUM_CORES + lax.axis_index("c")
    base = wid * ROWS_PER_W
    pltpu.sync_copy(idx_hbm.at[pl.ds(base, ROWS_PER_W)], idx_v)
    for c in range(NCHUNK):
        pltpu.async_copy(
            table_hbm.at[idx_v.at[pl.ds(c * GCHUNK, GCHUNK)]],
            rows_v.at[pl.ds(c * GCHUNK, GCHUNK)],
            sem,
        )
    for c in range(NCHUNK):
        pltpu.make_async_copy(
            table_hbm.at[idx_v.at[pl.ds(c * GCHUNK, GCHUNK)]],
            rows_v.at[pl.ds(c * GCHUNK, GCHUNK)],
            sem,
        ).wait()

    inv_w = jnp.float32(1.0 / WINDOW)

    def outer(b, carry):
        def inner(w, acc):
            r = b * WINDOW + w
            v = rows_v[r, pl.ds(0, 16)]
            ss = jnp.broadcast_to(jnp.sum(v * v), (16,))
            rs = _rsqrt16(ss)
            norm = ss * rs
            scale = jnp.where(ss > jnp.float32(1.0),
                              jnp.float32(1.0) / (norm + jnp.float32(1e-7)),
                              jnp.float32(1.0))
            return acc + v * scale

        acc = lax.fori_loop(0, WINDOW, inner, jnp.zeros((16,), jnp.float32))
        pool_v[b, pl.ds(0, 16)] = acc * inv_w
        return carry

    lax.fori_loop(0, B_PER_W, outer, 0)
    pltpu.sync_copy(pool_v, out_hbm.at[pl.ds(wid * B_PER_W, B_PER_W)])


@functools.cache
def _sc_pool():
    return pl.kernel(
        _sc_pool_body,
        mesh=plsc.VectorSubcoreMesh(core_axis_name="c", subcore_axis_name="s"),
        out_type=jax.ShapeDtypeStruct((BATCH, LANES), jnp.float32),
        scratch_types=[
            pltpu.VMEM((ROWS_PER_W,), jnp.int32),
            pltpu.VMEM((ROWS_PER_W, LANES), jnp.float32),
            pltpu.VMEM((B_PER_W, LANES), jnp.float32),
            pltpu.SemaphoreType.DMA,
        ],
        compiler_params=pltpu.CompilerParams(needs_layout_passes=False),
    )


def _mm_body(p_ref, w_ref, b_ref, o_ref):
    o_ref[...] = lax.dot_general(
        p_ref[:, :EMBED], w_ref[...],
        dimension_numbers=(((1,), (1,)), ((), ())),
        preferred_element_type=jnp.float32,
    ) + b_ref[...]


def _tc_project(pooled, W, b2):
    return pl.pallas_call(
        _mm_body,
        grid=(VGRID,),
        in_specs=[
            pl.BlockSpec((BATCH, LANES), lambda j: (0, 0)),
            pl.BlockSpec((VBLK, EMBED), lambda j: (j, 0)),
            pl.BlockSpec((1, VBLK), lambda j: (0, j)),
        ],
        out_specs=pl.BlockSpec((BATCH, VBLK), lambda j: (0, j)),
        out_shape=jax.ShapeDtypeStruct((BATCH, VOCAB), jnp.float32),
        compiler_params=pltpu.CompilerParams(
            dimension_semantics=("parallel",),
        ),
    )(pooled, W, b2)


def kernel(x, table, W, b):
    idx = x.reshape(-1).astype(jnp.int32)
    table_p = jnp.pad(table, ((0, 0), (0, LANES - EMBED)))
    pooled = _sc_pool()(idx, table_p)
    return _tc_project(pooled, W, b.reshape(1, VOCAB))


# speedup vs baseline: 2.9488x; 2.9488x over previous
import functools

import jax
import jax.numpy as jnp
from jax import lax
from jax.experimental import pallas as pl
from jax.experimental.pallas import tpu as pltpu
from jax.experimental.pallas import tpu_sc as plsc

VOCAB = 100000
EMBED = 16
WINDOW = 20
BATCH = 1024
LANES = 128

NUM_CORES = 2
NUM_SUBCORES = 16
NW = NUM_CORES * NUM_SUBCORES
B_PER_W = BATCH // NW
ROWS_PER_W = B_PER_W * WINDOW
GCHUNK = 128
NCHUNK = ROWS_PER_W // GCHUNK

VBLK = 2048
VGRID = (VOCAB + VBLK - 1) // VBLK


def _rsqrt16(s):
    s = jnp.maximum(s, jnp.float32(1e-12))
    i = lax.bitcast_convert_type(s, jnp.int32)
    i = jnp.int32(0x5F3759DF) - (i >> 1)
    y = lax.bitcast_convert_type(i, jnp.float32)
    for _ in range(3):
        y = y * (jnp.float32(1.5) - jnp.float32(0.5) * s * y * y)
    return y


def _sc_pool_body(idx_hbm, table_hbm, out_hbm, idx_v, rows_v, pool_v, sem):
    wid = lax.axis_index("s") * NUM_CORES + lax.axis_index("c")
    base = wid * ROWS_PER_W
    pltpu.sync_copy(idx_hbm.at[pl.ds(base, ROWS_PER_W)], idx_v)
    for c in range(NCHUNK):
        pltpu.async_copy(
            table_hbm.at[idx_v.at[pl.ds(c * GCHUNK, GCHUNK)]],
            rows_v.at[pl.ds(c * GCHUNK, GCHUNK)],
            sem,
        )
    for c in range(NCHUNK):
        pltpu.make_async_copy(
            table_hbm.at[idx_v.at[pl.ds(c * GCHUNK, GCHUNK)]],
            rows_v.at[pl.ds(c * GCHUNK, GCHUNK)],
            sem,
        ).wait()

    inv_w = jnp.float32(1.0 / WINDOW)
    one_lane = jnp.where(lax.iota(jnp.int32, 16) == 0,
                         jnp.float32(1.0), jnp.float32(0.0))

    def outer(b, carry):
        def inner(w, acc):
            r = b * WINDOW + w
            v = rows_v[r, pl.ds(0, 16)]
            ss = jnp.broadcast_to(jnp.sum(v * v), (16,))
            rs = _rsqrt16(ss)
            norm = ss * rs
            scale = jnp.where(ss > jnp.float32(1.0),
                              jnp.float32(1.0) / (norm + jnp.float32(1e-7)),
                              jnp.float32(1.0))
            return acc + v * scale

        acc = lax.fori_loop(0, WINDOW, inner, jnp.zeros((16,), jnp.float32))
        pool_v[b, pl.ds(0, 16)] = acc * inv_w
        pool_v[b, pl.ds(16, 16)] = one_lane
        return carry

    lax.fori_loop(0, B_PER_W, outer, 0)
    pltpu.sync_copy(pool_v, out_hbm.at[pl.ds(wid * B_PER_W, B_PER_W)])


@functools.cache
def _sc_pool():
    return pl.kernel(
        _sc_pool_body,
        mesh=plsc.VectorSubcoreMesh(core_axis_name="c", subcore_axis_name="s"),
        out_type=jax.ShapeDtypeStruct((BATCH, LANES), jnp.float32),
        scratch_types=[
            pltpu.VMEM((ROWS_PER_W,), jnp.int32),
            pltpu.VMEM((ROWS_PER_W, LANES), jnp.float32),
            pltpu.VMEM((B_PER_W, LANES), jnp.float32),
            pltpu.SemaphoreType.DMA,
        ],
        compiler_params=pltpu.CompilerParams(needs_layout_passes=False),
    )


def _mm_body(wt_ref, p_ref, b_ref, o_ref):
    wk = jnp.concatenate([wt_ref[...], b_ref[...]], axis=0)
    o_ref[...] = lax.dot_general(
        wk, p_ref[:, :EMBED + 1],
        dimension_numbers=(((0,), (1,)), ((), ())),
        preferred_element_type=jnp.float32,
    )


def _tc_project(W_t, pooled, b2):
    return pl.pallas_call(
        _mm_body,
        grid=(VGRID,),
        in_specs=[
            pl.BlockSpec((EMBED, VBLK), lambda j: (0, j)),
            pl.BlockSpec((BATCH, LANES), lambda j: (0, 0)),
            pl.BlockSpec((1, VBLK), lambda j: (0, j)),
        ],
        out_specs=pl.BlockSpec((VBLK, BATCH), lambda j: (j, 0)),
        out_shape=jax.ShapeDtypeStruct((VOCAB, BATCH), jnp.float32),
        compiler_params=pltpu.CompilerParams(
            dimension_semantics=("parallel",),
            fuse_transposed_lhs_in_matmul=True,
        ),
    )(W_t, pooled, b2)


def kernel(x, table, W, b):
    idx = x.reshape(-1).astype(jnp.int32)
    table_p = jnp.pad(table, ((0, 0), (0, LANES - EMBED)))
    pooled = _sc_pool()(idx, table_p)
    out_t = _tc_project(W.T, pooled, b.reshape(1, VOCAB))
    return out_t.T


# speedup vs baseline: 3.2678x; 1.1082x over previous
import functools

import jax
import jax.numpy as jnp
from jax import lax
from jax.experimental import pallas as pl
from jax.experimental.pallas import tpu as pltpu
from jax.experimental.pallas import tpu_sc as plsc

VOCAB = 100000
EMBED = 16
WINDOW = 20
BATCH = 1024
LANES = 128

NUM_CORES = 2
NUM_SUBCORES = 16
NW = NUM_CORES * NUM_SUBCORES
B_PER_W = BATCH // NW

WBLK = 4096
WGRID = (VOCAB + WBLK - 1) // WBLK
VBLK = 2048
VGRID = (VOCAB + VBLK - 1) // VBLK


def _prep_body(tt_ref, o_ref):
    t = tt_ref[...]
    ssq = jnp.sum(t * t, axis=0, keepdims=True)
    norm = jnp.sqrt(ssq)
    scale = jnp.where(ssq > jnp.float32(1.0),
                      jnp.float32(1.0) / (norm + jnp.float32(1e-7)),
                      jnp.float32(1.0))
    o_ref[:, :EMBED] = (t * scale).T


def _tc_prep(table_t):
    return pl.pallas_call(
        _prep_body,
        grid=(WGRID,),
        in_specs=[pl.BlockSpec((EMBED, WBLK), lambda j: (0, j))],
        out_specs=pl.BlockSpec((WBLK, LANES), lambda j: (j, 0)),
        out_shape=jax.ShapeDtypeStruct((VOCAB, LANES), jnp.float32),
        compiler_params=pltpu.CompilerParams(
            dimension_semantics=("parallel",),
        ),
    )(table_t)


def _sc_pool_body(idx_hbm, table_hbm, out_hbm, idx_v, rows_v, pool_v,
                  sem0, sem1):
    wid = lax.axis_index("s") * NUM_CORES + lax.axis_index("c")

    for w in range(WINDOW):
        pltpu.async_copy(idx_hbm.at[pl.ds(w * BATCH + wid * B_PER_W, B_PER_W)],
                         idx_v.at[pl.ds(w * B_PER_W, B_PER_W)], sem0)
    for w in range(WINDOW):
        pltpu.make_async_copy(
            idx_hbm.at[pl.ds(w * BATCH + wid * B_PER_W, B_PER_W)],
            idx_v.at[pl.ds(w * B_PER_W, B_PER_W)], sem0).wait()

    for c in range(WINDOW * B_PER_W // 128):
        pltpu.async_copy(table_hbm.at[idx_v.at[pl.ds(c * 128, 128)]],
                         rows_v.at[pl.ds(c * 128, 128)], sem1)
    for c in range(WINDOW * B_PER_W // 128):
        pltpu.make_async_copy(table_hbm.at[idx_v.at[pl.ds(c * 128, 128)]],
                              rows_v.at[pl.ds(c * 128, 128)], sem1).wait()

    def outer(b, carry):
        def inner(w, acc):
            return acc + rows_v[w * B_PER_W + b, pl.ds(0, 16)]

        acc = lax.fori_loop(0, WINDOW, inner, jnp.zeros((16,), jnp.float32))
        pool_v[b, pl.ds(0, 16)] = acc
        return carry

    lax.fori_loop(0, B_PER_W, outer, 0)
    pltpu.sync_copy(pool_v, out_hbm.at[pl.ds(wid * B_PER_W, B_PER_W)])


@functools.cache
def _sc_pool():
    return pl.kernel(
        _sc_pool_body,
        mesh=plsc.VectorSubcoreMesh(core_axis_name="c", subcore_axis_name="s"),
        out_type=jax.ShapeDtypeStruct((BATCH, LANES), jnp.float32),
        scratch_types=[
            pltpu.VMEM((B_PER_W * WINDOW,), jnp.int32),
            pltpu.VMEM((B_PER_W * WINDOW, LANES), jnp.float32),
            pltpu.VMEM((B_PER_W, LANES), jnp.float32),
            pltpu.SemaphoreType.DMA,
            pltpu.SemaphoreType.DMA,
        ],
        compiler_params=pltpu.CompilerParams(needs_layout_passes=False),
    )


def _mm_body(wt_ref, p_ref, b_ref, o_ref):
    wk = jnp.concatenate([wt_ref[...], b_ref[...]], axis=0)
    p17 = jnp.concatenate(
        [p_ref[:, :EMBED] * jnp.float32(1.0 / WINDOW),
         jnp.ones((BATCH, 1), jnp.float32)], axis=1)
    o_ref[...] = lax.dot_general(
        wk, p17,
        dimension_numbers=(((0,), (1,)), ((), ())),
        preferred_element_type=jnp.float32,
    )


def _tc_project(W_t, pooled, b2):
    return pl.pallas_call(
        _mm_body,
        grid=(VGRID,),
        in_specs=[
            pl.BlockSpec((EMBED, VBLK), lambda j: (0, j)),
            pl.BlockSpec((BATCH, LANES), lambda j: (0, 0)),
            pl.BlockSpec((1, VBLK), lambda j: (0, j)),
        ],
        out_specs=pl.BlockSpec((VBLK, BATCH), lambda j: (j, 0)),
        out_shape=jax.ShapeDtypeStruct((VOCAB, BATCH), jnp.float32),
        compiler_params=pltpu.CompilerParams(
            dimension_semantics=("parallel",),
            fuse_transposed_lhs_in_matmul=True,
        ),
    )(W_t, pooled, b2)


def kernel(x, table, W, b):
    idx_t = x.T.reshape(-1).astype(jnp.int32)
    table_sw = _tc_prep(table.T)
    pooled = _sc_pool()(idx_t, table_sw)
    out_t = _tc_project(W.T, pooled, b.reshape(1, VOCAB))
    return out_t.T
